# Initial kernel scaffold; baseline (speedup 1.0000x reference)
import functools

import jax
import jax.numpy as jnp
from jax import lax
from jax.experimental import pallas as pl
from jax.experimental.pallas import tpu as pltpu
from jax.experimental.pallas import tpu_sc as plsc


def _gather_rows(table, idx_flat, n_per_w, chunk, num_cores):
    n = idx_flat.shape[0]
    d = table.shape[1]
    n_chunks = n_per_w // chunk

    mesh = plsc.VectorSubcoreMesh(core_axis_name="c", subcore_axis_name="s")

    @functools.partial(
        pl.kernel,
        mesh=mesh,
        out_type=jax.ShapeDtypeStruct((n, d), jnp.float32),
        scratch_types=[
            pltpu.VMEM((n_per_w,), jnp.int32),
            pltpu.VMEM((chunk, d), jnp.float32),
            pltpu.SemaphoreType.DMA,
        ],
    )
    def k(table_hbm, idx_hbm, out_hbm, idx_v, rows_v, gsem):
        wid = lax.axis_index("s") * num_cores + lax.axis_index("c")
        base = wid * n_per_w
        pltpu.sync_copy(idx_hbm.at[pl.ds(base, n_per_w)], idx_v)

        def body(g, carry):
            off = g * chunk
            pltpu.async_copy(
                table_hbm.at[idx_v.at[pl.ds(off, chunk)]], rows_v, gsem
            ).wait()
            pltpu.sync_copy(rows_v, out_hbm.at[pl.ds(base + off, chunk)])
            return carry

        lax.fori_loop(0, n_chunks, body, 0)

    return k(table, idx_flat)


def kernel(rec_current, visited_time, pattern):
    b, s = visited_time.shape
    d = pattern.shape[1]
    n = b * s
    info = plsc.get_sparse_core_info()
    nw = info.num_cores * info.num_subcores
    n_per_w = n // nw
    idx_flat = visited_time.reshape(n)
    out = _gather_rows(pattern, idx_flat, n_per_w, 512, info.num_cores)
    return out.reshape(b, s, d)


# speedup vs baseline: 2.7810x; 2.7810x over previous
import functools

import jax
import jax.numpy as jnp
from jax import lax
from jax.experimental import pallas as pl
from jax.experimental.pallas import tpu as pltpu
from jax.experimental.pallas import tpu_sc as plsc


def _gather_rows(table, idx_flat, n_per_w, chunk, num_cores):
    n = idx_flat.shape[0]
    d = table.shape[1]
    n_chunks = n_per_w // chunk

    mesh = plsc.VectorSubcoreMesh(core_axis_name="c", subcore_axis_name="s")

    @functools.partial(
        pl.kernel,
        mesh=mesh,
        compiler_params=pltpu.CompilerParams(use_tc_tiling_on_sc=False),
        out_type=jax.ShapeDtypeStruct((n, d), jnp.float32),
        scratch_types=[
            pltpu.VMEM((n_per_w,), jnp.int32),
            pltpu.VMEM((chunk, d), jnp.float32),
            pltpu.SemaphoreType.DMA,
        ],
    )
    def k(table_hbm, idx_hbm, out_hbm, idx_v, rows_v, gsem):
        wid = lax.axis_index("s") * num_cores + lax.axis_index("c")
        base = wid * n_per_w
        pltpu.sync_copy(idx_hbm.at[pl.ds(base, n_per_w)], idx_v)

        def body(g, carry):
            off = g * chunk
            pltpu.async_copy(
                table_hbm.at[idx_v.at[pl.ds(off, chunk)]], rows_v, gsem
            ).wait()
            pltpu.sync_copy(rows_v, out_hbm.at[pl.ds(base + off, chunk)])
            return carry

        lax.fori_loop(0, n_chunks, body, 0)

    return k(table, idx_flat)


def kernel(rec_current, visited_time, pattern):
    b, s = visited_time.shape
    d = pattern.shape[1]
    n = b * s
    info = plsc.get_sparse_core_info()
    nw = info.num_cores * info.num_subcores
    n_per_w = n // nw
    idx_flat = visited_time.reshape(n)
    out = _gather_rows(pattern, idx_flat, n_per_w, 512, info.num_cores)
    return out.reshape(b, s, d)


# speedup vs baseline: 3.3713x; 1.2123x over previous
import functools

import jax
import jax.numpy as jnp
from jax import lax
from jax.experimental import pallas as pl
from jax.experimental.pallas import tpu as pltpu
from jax.experimental.pallas import tpu_sc as plsc


def _gather_rows(table_flat, idx_flat, n_per_w, chunk, num_cores, d):
    n = idx_flat.shape[0]
    n_chunks = n_per_w // chunk
    n_groups = n_chunks // 2
    table_words = table_flat.shape[0]
    cwords = chunk * d

    mesh = plsc.VectorSubcoreMesh(core_axis_name="c", subcore_axis_name="s")

    @functools.partial(
        pl.kernel,
        mesh=mesh,
        compiler_params=pltpu.CompilerParams(use_tc_tiling_on_sc=False),
        out_type=jax.ShapeDtypeStruct((n * d,), jnp.float32),
        scratch_types=[
            pltpu.VMEM((table_words,), jnp.float32),
            pltpu.VMEM((n_per_w,), jnp.int32),
            pltpu.VMEM((cwords,), jnp.float32),
            pltpu.VMEM((cwords,), jnp.float32),
            pltpu.SemaphoreType.DMA,
            pltpu.SemaphoreType.DMA,
        ],
    )
    def k(table_hbm, idx_hbm, out_hbm, table_v, idx_v, ob0, ob1, sem0, sem1):
        wid = lax.axis_index("s") * num_cores + lax.axis_index("c")
        base = wid * n_per_w
        pltpu.sync_copy(table_hbm, table_v)
        pltpu.sync_copy(idx_hbm.at[pl.ds(base, n_per_w)], idx_v)
        obufs = (ob0, ob1)
        sems = (sem0, sem1)

        def expand(off, obuf):
            def blk(t, c):
                i0 = t * 16
                bvec = idx_v[pl.ds(off + i0, 16)] * d
                for r in range(16):
                    b = bvec[r]
                    for q in range(d // 16):
                        obuf[pl.ds((i0 + r) * d + q * 16, 16)] = table_v[
                            pl.ds(b + q * 16, 16)
                        ]
                return c

            lax.fori_loop(0, chunk // 16, blk, 0, unroll=2)

        def wait_scatter(j):
            pltpu.make_async_copy(
                obufs[j], out_hbm.at[pl.ds(0, cwords)], sems[j]
            ).wait()

        def group(p, c):
            for j in range(2):
                off = (p * 2 + j) * chunk

                @pl.when(p > 0)
                def _():
                    wait_scatter(j)

                expand(off, obufs[j])
                pltpu.async_copy(
                    obufs[j],
                    out_hbm.at[pl.ds((base + off) * d, cwords)],
                    sems[j],
                )
            return c

        lax.fori_loop(0, n_groups, group, 0)
        wait_scatter(0)
        wait_scatter(1)

    return k(table_flat, idx_flat)


def kernel(rec_current, visited_time, pattern):
    b, s = visited_time.shape
    d = pattern.shape[1]
    n = b * s
    info = plsc.get_sparse_core_info()
    nw = info.num_cores * info.num_subcores
    n_per_w = n // nw
    idx_flat = visited_time.reshape(n)
    out = _gather_rows(
        pattern.reshape(-1), idx_flat, n_per_w, 512, info.num_cores, d
    )
    return out.reshape(b, s, d)
